# baseline (speedup 1.0000x reference)
import functools

import jax
import jax.numpy as jnp
from jax import lax
from jax.experimental import pallas as pl
from jax.experimental.pallas import tpu as pltpu
from jax.experimental.pallas import tpu_sc as plsc

C, H, W, S = 96, 512, 512, 128
ROWS = C * H
NTILES = 32
RPT = ROWS // NTILES
CH = 96
NCH = RPT // CH
JG = 8

_sc_mesh = plsc.VectorSubcoreMesh(core_axis_name="c", subcore_axis_name="s")


@functools.partial(
    pl.kernel,
    mesh=_sc_mesh,
    out_type=jax.ShapeDtypeStruct((ROWS, W), jnp.float32),
    scratch_types=[
        pltpu.VMEM((S,), jnp.int32),
        pltpu.VMEM((W,), jnp.float32),
        pltpu.VMEM((W,), jnp.float32),
        pltpu.VMEM((S,), jnp.float32),
        pltpu.VMEM((CH, W), jnp.float32),
        pltpu.VMEM((CH, W), jnp.float32),
        pltpu.VMEM_SHARED((W,), jnp.float32),
        pltpu.VMEM_SHARED((W,), jnp.float32),
        pltpu.SemaphoreType.DMA,
        pltpu.SemaphoreType.DMA,
        pltpu.SemaphoreType.DMA,
        pltpu.SemaphoreType.DMA,
    ],
)
def _sc_flip(x_hbm, sites_hbm, o_hbm,
             sites_v, a_v, b_v, vals_v, buf0, buf1, a_sh, b_sh,
             sin0, sin1, sout0, sout1):
    cid = lax.axis_index("c")
    sid = lax.axis_index("s")
    wid = sid * 2 + cid

    @pl.when(sid == 0)
    def _build():
        ones = jnp.full((16,), 1.0, jnp.float32)
        zeros = jnp.zeros((16,), jnp.float32)
        neg = jnp.full((16,), -1.0, jnp.float32)
        pltpu.sync_copy(sites_hbm, sites_v)
        for i in range(W // 16):
            a_v[pl.ds(i * 16, 16)] = ones
            b_v[pl.ds(i * 16, 16)] = zeros
        pltpu.sync_copy(a_v, a_sh)
        pltpu.sync_copy(b_v, b_sh)
        for j in range(S // 16):
            vals_v[pl.ds(j * 16, 16)] = neg
        pltpu.sync_copy(vals_v, a_sh.at[sites_v])
        for j in range(S // 16):
            vals_v[pl.ds(j * 16, 16)] = ones
        pltpu.sync_copy(vals_v, b_sh.at[sites_v])

    plsc.subcore_barrier()
    pltpu.sync_copy(a_sh, a_v)
    pltpu.sync_copy(b_sh, b_v)

    bufs = (buf0, buf1)
    sins = (sin0, sin1)
    souts = (sout0, sout1)
    base = wid * RPT

    def _compute(buf):
        for jg in range(W // (16 * JG)):
            avs = [a_v[pl.ds((jg * JG + k) * 16, 16)] for k in range(JG)]
            bvs = [b_v[pl.ds((jg * JG + k) * 16, 16)] for k in range(JG)]

            def _row(r, carry):
                for k in range(JG):
                    sl = pl.ds((jg * JG + k) * 16, 16)
                    buf[r, sl] = buf[r, sl] * avs[k] + bvs[k]
                return carry

            lax.fori_loop(0, CH, _row, 0)

    in_cp = [None] * NCH
    out_cp = [None] * NCH
    in_cp[0] = pltpu.async_copy(
        x_hbm.at[pl.ds(base, CH)], bufs[0], sins[0])
    for g in range(NCH):
        b = g % 2
        in_cp[g].wait()
        if g + 1 < NCH:
            if g >= 1:
                out_cp[g - 1].wait()
            nb = (g + 1) % 2
            in_cp[g + 1] = pltpu.async_copy(
                x_hbm.at[pl.ds(base + (g + 1) * CH, CH)], bufs[nb], sins[nb])
        _compute(bufs[b])
        out_cp[g] = pltpu.async_copy(
            bufs[b], o_hbm.at[pl.ds(base + g * CH, CH)], souts[b])
    out_cp[NCH - 1].wait()


def kernel(x, mask_sites):
    x2 = x.reshape(ROWS, W)
    out = _sc_flip(x2, mask_sites)
    return out.reshape(C, H, W)


# speedup vs baseline: 1.2332x; 1.2332x over previous
import functools

import jax
import jax.numpy as jnp
from jax import lax
from jax.experimental import pallas as pl
from jax.experimental.pallas import tpu as pltpu
from jax.experimental.pallas import tpu_sc as plsc

C, H, W, S = 96, 512, 512, 128
R_BLK = 6144

_sc_mesh = plsc.VectorSubcoreMesh(
    core_axis_name="c", subcore_axis_name="s", num_cores=1)


@functools.partial(
    pl.kernel,
    mesh=_sc_mesh,
    out_type=jax.ShapeDtypeStruct((W,), jnp.float32),
    scratch_types=[
        pltpu.VMEM((S,), jnp.int32),
        pltpu.VMEM((W,), jnp.float32),
        pltpu.VMEM((S,), jnp.float32),
        pltpu.VMEM_SHARED((W,), jnp.float32),
        pltpu.SemaphoreType.DMA,
        pltpu.SemaphoreType.DMA,
    ],
)
def _build_mask(sites_hbm, m_hbm, sites_v, zeros_v, ones_v, m_sh, sem1, sem2):
    wid = lax.axis_index("s")

    @pl.when(wid == 0)
    def _():
        ones = jnp.full((16,), 1.0, jnp.float32)
        zeros = jnp.zeros((16,), jnp.float32)
        for i in range(W // 16):
            zeros_v[pl.ds(i * 16, 16)] = zeros
        for j in range(S // 16):
            ones_v[pl.ds(j * 16, 16)] = ones
        cp_sites = pltpu.async_copy(sites_hbm, sites_v, sem1)
        cp_init = pltpu.async_copy(zeros_v, m_sh, sem2)
        cp_sites.wait()
        cp_init.wait()
        pltpu.sync_copy(ones_v, m_sh.at[sites_v])
        pltpu.sync_copy(m_sh, m_hbm)


def _flip_body(m_ref, x_ref, o_ref, a_ref, b_ref):
    @pl.when(pl.program_id(0) == 0)
    def _build_tables():
        m = m_ref[...]
        a_ref[...] = jnp.broadcast_to(1.0 - 2.0 * m, (8, W))
        b_ref[...] = jnp.broadcast_to(m, (8, W))

    rep = R_BLK // 8
    o_ref[...] = x_ref[...] * jnp.tile(a_ref[...], (rep, 1)) + jnp.tile(
        b_ref[...], (rep, 1))


def kernel(x, mask_sites):
    m = _build_mask(mask_sites)
    x2 = x.reshape(C * H, W)
    out = pl.pallas_call(
        _flip_body,
        grid=((C * H) // R_BLK,),
        in_specs=[
            pl.BlockSpec((1, W), lambda i: (0, 0)),
            pl.BlockSpec((R_BLK, W), lambda i: (i, 0)),
        ],
        out_specs=pl.BlockSpec((R_BLK, W), lambda i: (i, 0)),
        out_shape=jax.ShapeDtypeStruct((C * H, W), jnp.float32),
        scratch_shapes=[
            pltpu.VMEM((8, W), jnp.float32),
            pltpu.VMEM((8, W), jnp.float32),
        ],
    )(m.reshape(1, W), x2)
    return out.reshape(C, H, W)
